# Initial kernel scaffold; baseline (speedup 1.0000x reference)
import jax
import jax.numpy as jnp
from jax.experimental import pallas as pl


def kernel(inputs, segment_ids, W1, b1, W2, b2, W3, b3, Wr, br, Wk, bk, Wq):
    raise NotImplementedError("write your pallas kernel here")


# speedup vs baseline: 4.6701x; 4.6701x over previous
import jax
import jax.numpy as jnp
from jax.experimental import pallas as pl
from jax.experimental.pallas import tpu as pltpu

HIGHEST = jax.lax.Precision.HIGHEST

N = 32768
D_IN = 32
WIDTH = 128
LATENT = 128
N_HEADS = 4
DP = 64
NUM_SEG = 16

R = 8192
T = 128
NB = N // R
NC = R // T


def _split_hi_lo(v):
    hi = v.astype(jnp.bfloat16).astype(jnp.float32)
    return hi, v - hi


def _mask_matmul(mask, vals):
    hi, lo = _split_hi_lo(vals)
    return jnp.matmul(mask, hi) + jnp.matmul(mask, lo)


def _mask_dot_t(mask, vals):
    dn = (((0,), (0,)), ((), ()))
    hi, lo = _split_hi_lo(vals)
    return (jax.lax.dot_general(mask, hi, dn)
            + jax.lax.dot_general(mask, lo, dn))


def _matmul3(a, b):
    a_hi, a_lo = _split_hi_lo(a)
    b_hi, b_lo = _split_hi_lo(b)
    return a_hi @ b_hi + (a_hi @ b_lo + a_lo @ b_hi)


def _body(x_ref, ids_ref, W1_ref, b1_ref, W2_ref, b2_ref, W3_ref, b3_ref,
          Wr_ref, br_ref, Wk_ref, bk_ref, Wq_ref, out_ref,
          A_s, C_s, c_s, ltsum_s, ltcnt_s, tot_s):
    k = pl.program_id(0)

    @pl.when(k == 0)
    def _fold():
        scale = 1.0 / (DP ** 0.5)
        vcols = []
        ckcols = []
        for h in range(N_HEADS):
            wk_h = Wk_ref[:, h * DP:(h + 1) * DP]
            wq_h = Wq_ref[h:h + 1, :]
            vcols.append(jax.lax.dot_general(
                wk_h, wq_h, (((1,), (1,)), ((), ())),
                precision=HIGHEST))
            ckcols.append(jax.lax.dot_general(
                bk_ref[:, h * DP:(h + 1) * DP], wq_h,
                (((1,), (1,)), ((), ())), precision=HIGHEST))
        V = jnp.concatenate(vcols, axis=1) * scale
        ck = jnp.concatenate(ckcols, axis=1) * scale
        A = V[:D_IN, :]
        V2 = V[D_IN:, :]
        B = jnp.matmul(Wr_ref[...], V2, precision=HIGHEST)
        C = jnp.matmul(W3_ref[...], B, precision=HIGHEST)
        const = (jnp.matmul(b3_ref[...], B, precision=HIGHEST)
                 + jnp.matmul(br_ref[...], V2, precision=HIGHEST) + ck)
        A_s[...] = A
        C_s[...] = C
        c_s[...] = const
        ltsum_s[...] = jnp.zeros((NUM_SEG, N_HEADS), jnp.float32)
        ltcnt_s[...] = jnp.zeros((1, NUM_SEG), jnp.float32)
        tot_s[...] = jnp.zeros((1, N_HEADS), jnp.float32)

    x = x_ref[...]
    h1 = jnp.maximum(_matmul3(x, W1_ref[...]) + b1_ref[...], 0.0)
    h2 = jnp.maximum(_matmul3(h1, W2_ref[...]) + b2_ref[...], 0.0)
    z = _matmul3(h2, C_s[...])

    tri = (jax.lax.broadcasted_iota(jnp.int32, (T, T), 0)
           >= jax.lax.broadcasted_iota(jnp.int32, (T, T), 1)).astype(jnp.float32)
    off = tot_s[...]
    parts = []
    for c in range(NC):
        zc = z[c * T:(c + 1) * T, :]
        parts.append(_mask_matmul(tri, zc) + off)
        off = off + jnp.sum(zc, axis=0, keepdims=True)
    csum = jnp.concatenate(parts, axis=0)

    ids = ids_ref[...]
    srange = jax.lax.broadcasted_iota(jnp.int32, (1, NUM_SEG), 1)
    eq = (ids == srange).astype(jnp.float32)

    seg_sum = _mask_dot_t(eq, z)
    seg_cnt = jnp.sum(eq, axis=0, keepdims=True)
    stri = (jax.lax.broadcasted_iota(jnp.int32, (NUM_SEG, NUM_SEG), 0)
            > jax.lax.broadcasted_iota(jnp.int32, (NUM_SEG, NUM_SEG), 1)
            ).astype(jnp.float32)
    loc_lt_sum = jnp.matmul(stri, seg_sum, precision=HIGHEST)
    loc_lt_cnt = jnp.matmul(seg_cnt, jax.lax.transpose(stri, (1, 0)),
                            precision=HIGHEST)

    lt_tot = ltsum_s[...] + loc_lt_sum
    cnt_col = jax.lax.transpose(ltcnt_s[...] + loc_lt_cnt, (1, 0))
    sel_tab = jnp.concatenate([lt_tot, cnt_col], axis=1)
    sel = _mask_matmul(eq, sel_tab)
    base = sel[:, :N_HEADS]
    start = jnp.round(sel[:, N_HEADS:N_HEADS + 1])
    gidx = (k * R + jax.lax.broadcasted_iota(jnp.int32, (R, 1), 0)).astype(jnp.float32)
    pos = gidx + 1.0 - start

    cm = (csum - base) / pos
    out_ref[...] = _matmul3(x, A_s[...]) + cm + c_s[...]

    tot_s[...] = off
    ltsum_s[...] = ltsum_s[...] + loc_lt_sum
    ltcnt_s[...] = ltcnt_s[...] + loc_lt_cnt


def kernel(inputs, segment_ids, W1, b1, W2, b2, W3, b3, Wr, br, Wk, bk, Wq):
    ids2d = segment_ids.astype(jnp.int32).reshape(N, 1)
    full = lambda shape: pl.BlockSpec(shape, lambda k: (0, 0))
    out = pl.pallas_call(
        _body,
        grid=(NB,),
        in_specs=[
            pl.BlockSpec((R, D_IN), lambda k: (k, 0)),
            pl.BlockSpec((R, 1), lambda k: (k, 0)),
            full((D_IN, WIDTH)),
            full((1, WIDTH)),
            full((WIDTH, WIDTH)),
            full((1, WIDTH)),
            full((WIDTH, LATENT)),
            full((1, LATENT)),
            full((LATENT, LATENT)),
            full((1, LATENT)),
            full((D_IN + LATENT, N_HEADS * DP)),
            full((1, N_HEADS * DP)),
            full((N_HEADS, DP)),
        ],
        out_specs=pl.BlockSpec((R, N_HEADS), lambda k: (k, 0)),
        out_shape=jax.ShapeDtypeStruct((N, N_HEADS), jnp.float32),
        scratch_shapes=[
            pltpu.VMEM((D_IN, N_HEADS), jnp.float32),
            pltpu.VMEM((LATENT, N_HEADS), jnp.float32),
            pltpu.VMEM((1, N_HEADS), jnp.float32),
            pltpu.VMEM((NUM_SEG, N_HEADS), jnp.float32),
            pltpu.VMEM((1, NUM_SEG), jnp.float32),
            pltpu.VMEM((1, N_HEADS), jnp.float32),
        ],
    )(inputs, ids2d, W1, b1.reshape(1, WIDTH), W2, b2.reshape(1, WIDTH),
      W3, b3.reshape(1, LATENT), Wr, br.reshape(1, LATENT),
      Wk, bk.reshape(1, N_HEADS * DP), Wq)
    return out


# speedup vs baseline: 4.7260x; 1.0120x over previous
import jax
import jax.numpy as jnp
from jax.experimental import pallas as pl
from jax.experimental.pallas import tpu as pltpu

HIGHEST = jax.lax.Precision.HIGHEST

N = 32768
D_IN = 32
WIDTH = 128
LATENT = 128
N_HEADS = 4
DP = 64
NUM_SEG = 16

R = 8192
T = 128
NB = N // R
NC = R // T


def _split_hi_lo(v):
    b = jax.lax.bitcast_convert_type(v, jnp.uint32)
    hi = jax.lax.bitcast_convert_type(b & jnp.uint32(0xFFFF0000), jnp.float32)
    return hi, v - hi


def _mask_matmul(mask, vals):
    hi, lo = _split_hi_lo(vals)
    return jnp.matmul(mask, hi) + jnp.matmul(mask, lo)


def _mask_dot_t(mask, vals):
    dn = (((0,), (0,)), ((), ()))
    hi, lo = _split_hi_lo(vals)
    return (jax.lax.dot_general(mask, hi, dn)
            + jax.lax.dot_general(mask, lo, dn))


def _matmul3(a, b):
    a_hi, a_lo = _split_hi_lo(a)
    b_hi, b_lo = _split_hi_lo(b)
    return a_hi @ b_hi + (a_hi @ b_lo + a_lo @ b_hi)


def _matmul3_pre(a_hi, a_lo, b_hi, b_lo):
    return a_hi @ b_hi + (a_hi @ b_lo + a_lo @ b_hi)


def _body(x_ref, ids_ref, W1_ref, b1_ref, W2_ref, b2_ref, W3_ref, b3_ref,
          Wr_ref, br_ref, Wk_ref, bk_ref, Wq_ref, out_ref,
          A_s, C_s, c_s, ltsum_s, ltcnt_s, tot_s,
          W1h_s, W1l_s, W2h_s, W2l_s, Ch_s, Cl_s, Ah_s, Al_s, tri_s):
    k = pl.program_id(0)

    @pl.when(k == 0)
    def _fold():
        scale = 1.0 / (DP ** 0.5)
        vcols = []
        ckcols = []
        for h in range(N_HEADS):
            wk_h = Wk_ref[:, h * DP:(h + 1) * DP]
            wq_h = Wq_ref[h:h + 1, :]
            vcols.append(jax.lax.dot_general(
                wk_h, wq_h, (((1,), (1,)), ((), ())),
                precision=HIGHEST))
            ckcols.append(jax.lax.dot_general(
                bk_ref[:, h * DP:(h + 1) * DP], wq_h,
                (((1,), (1,)), ((), ())), precision=HIGHEST))
        V = jnp.concatenate(vcols, axis=1) * scale
        ck = jnp.concatenate(ckcols, axis=1) * scale
        A = V[:D_IN, :]
        V2 = V[D_IN:, :]
        B = jnp.matmul(Wr_ref[...], V2, precision=HIGHEST)
        C = jnp.matmul(W3_ref[...], B, precision=HIGHEST)
        const = (jnp.matmul(b3_ref[...], B, precision=HIGHEST)
                 + jnp.matmul(br_ref[...], V2, precision=HIGHEST) + ck)
        A_s[...] = A
        C_s[...] = C
        c_s[...] = const
        ltsum_s[...] = jnp.zeros((NUM_SEG, N_HEADS), jnp.float32)
        ltcnt_s[...] = jnp.zeros((1, NUM_SEG), jnp.float32)
        tot_s[...] = jnp.zeros((1, N_HEADS), jnp.float32)
        W1h_s[...], W1l_s[...] = _split_hi_lo(W1_ref[...])
        W2h_s[...], W2l_s[...] = _split_hi_lo(W2_ref[...])
        Ch_s[...], Cl_s[...] = _split_hi_lo(C)
        Ah_s[...], Al_s[...] = _split_hi_lo(A)
        tri_s[...] = (jax.lax.broadcasted_iota(jnp.int32, (T, T), 0)
                      >= jax.lax.broadcasted_iota(jnp.int32, (T, T), 1)
                      ).astype(jnp.float32)

    x = x_ref[...]
    xh, xl = _split_hi_lo(x)
    h1 = jnp.maximum(
        _matmul3_pre(xh, xl, W1h_s[...], W1l_s[...]) + b1_ref[...], 0.0)
    h1h, h1l = _split_hi_lo(h1)
    h2 = jnp.maximum(
        _matmul3_pre(h1h, h1l, W2h_s[...], W2l_s[...]) + b2_ref[...], 0.0)
    h2h, h2l = _split_hi_lo(h2)
    z = _matmul3_pre(h2h, h2l, Ch_s[...], Cl_s[...])

    tri = tri_s[...]
    off = tot_s[...]
    parts = []
    for c in range(NC):
        zc = z[c * T:(c + 1) * T, :]
        parts.append(_mask_matmul(tri, zc) + off)
        off = off + jnp.sum(zc, axis=0, keepdims=True)
    csum = jnp.concatenate(parts, axis=0)

    ids = ids_ref[...]
    srange = jax.lax.broadcasted_iota(jnp.int32, (1, NUM_SEG), 1)
    eq = (ids == srange).astype(jnp.float32)

    seg_sum = _mask_dot_t(eq, z)
    seg_cnt = jnp.sum(eq, axis=0, keepdims=True)
    stri = (jax.lax.broadcasted_iota(jnp.int32, (NUM_SEG, NUM_SEG), 0)
            > jax.lax.broadcasted_iota(jnp.int32, (NUM_SEG, NUM_SEG), 1)
            ).astype(jnp.float32)
    loc_lt_sum = jnp.matmul(stri, seg_sum, precision=HIGHEST)
    loc_lt_cnt = jnp.matmul(seg_cnt, jax.lax.transpose(stri, (1, 0)),
                            precision=HIGHEST)

    lt_tot = ltsum_s[...] + loc_lt_sum
    cnt_col = jax.lax.transpose(ltcnt_s[...] + loc_lt_cnt, (1, 0))
    sel_tab = jnp.concatenate([lt_tot, cnt_col], axis=1)
    sel = _mask_matmul(eq, sel_tab)
    base = sel[:, :N_HEADS]
    start = jnp.round(sel[:, N_HEADS:N_HEADS + 1])
    gidx = (k * R + jax.lax.broadcasted_iota(jnp.int32, (R, 1), 0)).astype(jnp.float32)
    pos = gidx + 1.0 - start

    cm = (csum - base) / pos
    out_ref[...] = _matmul3_pre(xh, xl, Ah_s[...], Al_s[...]) + cm + c_s[...]

    tot_s[...] = off
    ltsum_s[...] = ltsum_s[...] + loc_lt_sum
    ltcnt_s[...] = ltcnt_s[...] + loc_lt_cnt


def kernel(inputs, segment_ids, W1, b1, W2, b2, W3, b3, Wr, br, Wk, bk, Wq):
    ids2d = segment_ids.astype(jnp.int32).reshape(N, 1)
    full = lambda shape: pl.BlockSpec(shape, lambda k: (0, 0))
    out = pl.pallas_call(
        _body,
        grid=(NB,),
        in_specs=[
            pl.BlockSpec((R, D_IN), lambda k: (k, 0)),
            pl.BlockSpec((R, 1), lambda k: (k, 0)),
            full((D_IN, WIDTH)),
            full((1, WIDTH)),
            full((WIDTH, WIDTH)),
            full((1, WIDTH)),
            full((WIDTH, LATENT)),
            full((1, LATENT)),
            full((LATENT, LATENT)),
            full((1, LATENT)),
            full((D_IN + LATENT, N_HEADS * DP)),
            full((1, N_HEADS * DP)),
            full((N_HEADS, DP)),
        ],
        out_specs=pl.BlockSpec((R, N_HEADS), lambda k: (k, 0)),
        out_shape=jax.ShapeDtypeStruct((N, N_HEADS), jnp.float32),
        scratch_shapes=[
            pltpu.VMEM((D_IN, N_HEADS), jnp.float32),
            pltpu.VMEM((LATENT, N_HEADS), jnp.float32),
            pltpu.VMEM((1, N_HEADS), jnp.float32),
            pltpu.VMEM((NUM_SEG, N_HEADS), jnp.float32),
            pltpu.VMEM((1, NUM_SEG), jnp.float32),
            pltpu.VMEM((1, N_HEADS), jnp.float32),
            pltpu.VMEM((D_IN, WIDTH), jnp.float32),
            pltpu.VMEM((D_IN, WIDTH), jnp.float32),
            pltpu.VMEM((WIDTH, WIDTH), jnp.float32),
            pltpu.VMEM((WIDTH, WIDTH), jnp.float32),
            pltpu.VMEM((LATENT, N_HEADS), jnp.float32),
            pltpu.VMEM((LATENT, N_HEADS), jnp.float32),
            pltpu.VMEM((D_IN, N_HEADS), jnp.float32),
            pltpu.VMEM((D_IN, N_HEADS), jnp.float32),
            pltpu.VMEM((T, T), jnp.float32),
        ],
    )(inputs, ids2d, W1, b1.reshape(1, WIDTH), W2, b2.reshape(1, WIDTH),
      W3, b3.reshape(1, LATENT), Wr, br.reshape(1, LATENT),
      Wk, bk.reshape(1, N_HEADS * DP), Wq)
    return out
